# baseline (speedup 1.0000x reference)
import functools

import jax
import jax.numpy as jnp
from jax import lax
from jax.experimental import pallas as pl
from jax.experimental.pallas import tpu as pltpu
from jax.experimental.pallas import tpu_sc as plsc

_NC = 2
_NS = 16
_NW = _NC * _NS
_CHUNK = 128


@functools.lru_cache(maxsize=None)
def _make_gather(num_rows_total, dim, b_per_w):
    k = b_per_w // _CHUNK
    mesh = plsc.VectorSubcoreMesh(core_axis_name="c", subcore_axis_name="s")

    @functools.partial(
        pl.kernel,
        mesh=mesh,
        compiler_params=pltpu.CompilerParams(use_tc_tiling_on_sc=False),
        out_type=jax.ShapeDtypeStruct((num_rows_total, dim), jnp.float32),
        scratch_types=[
            pltpu.VMEM((k, _CHUNK), jnp.int32),
            pltpu.VMEM((_CHUNK, dim), jnp.float32),
            pltpu.SemaphoreType.DMA,
        ],
    )
    def body(idx_hbm, table_hbm, out_hbm, idx_v, rows_v, sem):
        wid = lax.axis_index("s") * _NC + lax.axis_index("c")
        base = wid * b_per_w
        pltpu.sync_copy(idx_hbm.at[wid], idx_v)

        def step(j, carry):
            pltpu.async_copy(table_hbm.at[idx_v.at[j]], rows_v, sem).wait()
            pltpu.sync_copy(rows_v, out_hbm.at[pl.ds(base + j * _CHUNK, _CHUNK)])
            return carry

        lax.fori_loop(0, k, step, 0)

    return body


def kernel(token_ids, weight):
    orig_shape = token_ids.shape
    b_total = token_ids.size
    b_per_w = b_total // _NW
    idx = token_ids.reshape(_NW, b_per_w // _CHUNK, _CHUNK).astype(jnp.int32)
    out = _make_gather(b_total, weight.shape[1], b_per_w)(idx, weight)
    return out.reshape(*orig_shape, weight.shape[1])


# speedup vs baseline: 1.0482x; 1.0482x over previous
import functools

import jax
import jax.numpy as jnp
from jax import lax
from jax.experimental import pallas as pl
from jax.experimental.pallas import tpu as pltpu
from jax.experimental.pallas import tpu_sc as plsc

_NC = 2
_NS = 16
_NW = _NC * _NS
_CHUNK = 128
_NBUF = 5


@functools.lru_cache(maxsize=None)
def _make_gather(num_rows_total, dim, b_per_w):
    k = b_per_w // _CHUNK
    assert k % _NBUF == 0
    mesh = plsc.VectorSubcoreMesh(core_axis_name="c", subcore_axis_name="s")

    @functools.partial(
        pl.kernel,
        mesh=mesh,
        compiler_params=pltpu.CompilerParams(use_tc_tiling_on_sc=False),
        out_type=jax.ShapeDtypeStruct((num_rows_total, dim), jnp.float32),
        scratch_types=(
            [pltpu.VMEM((k, _CHUNK), jnp.int32),
             pltpu.VMEM((_NBUF, _CHUNK, dim), jnp.float32)]
            + [pltpu.SemaphoreType.DMA] * (2 * _NBUF)
        ),
    )
    def body(idx_hbm, table_hbm, out_hbm, idx_v, rows_v, *sems):
        gsem = sems[:_NBUF]
        osem = sems[_NBUF:]
        wid = lax.axis_index("s") * _NC + lax.axis_index("c")
        base = wid * b_per_w
        pltpu.sync_copy(idx_hbm.at[wid], idx_v)

        def gather_desc(j, b):
            return pltpu.make_async_copy(
                table_hbm.at[idx_v.at[j]], rows_v.at[b], gsem[b])

        def out_desc(j, b):
            return pltpu.make_async_copy(
                rows_v.at[b], out_hbm.at[pl.ds(base + j * _CHUNK, _CHUNK)],
                osem[b])

        for b in range(_NBUF):
            gather_desc(b, b).start()

        @pl.loop(0, k, step=_NBUF)
        def group(g):
            for b in range(_NBUF):
                j = g + b
                gather_desc(j, b).wait()
                out_desc(j, b).start()
                out_desc(j, b).wait()

                @pl.when(j + _NBUF < k)
                def _():
                    gather_desc(j + _NBUF, b).start()

    return body


def kernel(token_ids, weight):
    orig_shape = token_ids.shape
    b_total = token_ids.size
    b_per_w = b_total // _NW
    idx = token_ids.reshape(_NW, b_per_w // _CHUNK, _CHUNK).astype(jnp.int32)
    out = _make_gather(b_total, weight.shape[1], b_per_w)(idx, weight)
    return out.reshape(*orig_shape, weight.shape[1])


# speedup vs baseline: 1.5517x; 1.4803x over previous
import functools

import jax
import jax.numpy as jnp
from jax import lax
from jax.experimental import pallas as pl
from jax.experimental.pallas import tpu as pltpu
from jax.experimental.pallas import tpu_sc as plsc

_NC = 2
_NS = 16
_NW = _NC * _NS
_CH = 4


@functools.lru_cache(maxsize=None)
def _make_gather(num_rows, dim, batch, seq):
    rows_per_w = batch // _NW
    n_chunks = rows_per_w // _CH
    mesh = plsc.VectorSubcoreMesh(core_axis_name="c", subcore_axis_name="s")

    @functools.partial(
        pl.kernel,
        mesh=mesh,
        out_type=jax.ShapeDtypeStruct((batch, seq, dim), jnp.float32),
        scratch_types=[
            pltpu.VMEM((rows_per_w * seq,), jnp.int32),
            pltpu.VMEM_SHARED((_NS * rows_per_w * seq,), jnp.int32),
            pltpu.SMEM((_CH * seq,), jnp.int32),
            pltpu.SMEM((_CH * seq,), jnp.int32),
            pltpu.VMEM((_CH, seq, dim), jnp.float32),
            pltpu.VMEM((_CH, seq, dim), jnp.float32),
            pltpu.SemaphoreType.DMA,
            pltpu.SemaphoreType.DMA,
            pltpu.SemaphoreType.DMA,
            pltpu.SemaphoreType.DMA,
        ],
    )
    def body(ids_hbm, table_hbm, out_hbm,
             ids_v, ids_sh, sm0, sm1, r0_v, r1_v, rsem0, rsem1, osem0, osem1):
        sm = (sm0, sm1)
        rbuf = (r0_v, r1_v)
        rsem = (rsem0, rsem1)
        osem = (osem0, osem1)
        sid = lax.axis_index("s")
        wid = sid * _NC + lax.axis_index("c")
        base = wid * rows_per_w
        cw = _CH * seq
        shbase = sid * (rows_per_w * seq)
        pltpu.sync_copy(ids_hbm.at[pl.ds(base * seq, rows_per_w * seq)], ids_v)
        pltpu.sync_copy(ids_v, ids_sh.at[pl.ds(shbase, rows_per_w * seq)])
        pltpu.sync_copy(ids_sh.at[pl.ds(shbase, cw)], sm0)
        pltpu.sync_copy(ids_sh.at[pl.ds(shbase + cw, cw)], sm1)

        def out_desc(c, b):
            return pltpu.make_async_copy(
                rbuf[b], out_hbm.at[pl.ds(base + c * _CH, _CH)], osem[b])

        def row_drain_desc(c, b):
            return pltpu.make_async_copy(
                out_hbm.at[pl.ds(base + c * _CH, _CH)], rbuf[b], rsem[b])

        @pl.loop(0, n_chunks, step=2)
        def grp(c0):
            for b in range(2):
                c = c0 + b

                @pl.when(c >= 2)
                def _():
                    out_desc(c - 2, b).wait()

                for j in range(_CH):
                    for t in range(seq):
                        tid = sm[b][j * seq + t]
                        pltpu.make_async_copy(
                            table_hbm.at[tid], rbuf[b].at[j, t], rsem[b]
                        ).start()

                @pl.when(c + 2 < n_chunks)
                def _():
                    pltpu.sync_copy(
                        ids_sh.at[pl.ds(shbase + (c + 2) * cw, cw)], sm[b])

                row_drain_desc(c, b).wait()
                out_desc(c, b).start()

        out_desc(n_chunks - 2, 0).wait()
        out_desc(n_chunks - 1, 1).wait()

    return body


def kernel(token_ids, weight):
    batch, seq = token_ids.shape
    num_rows, dim = weight.shape
    ids = token_ids.astype(jnp.int32).reshape(-1)
    return _make_gather(num_rows, dim, batch, seq)(ids, weight)
